# baseline (speedup 1.0000x reference)
import jax
from jax.experimental import pallas as pl
from jax.experimental.pallas import tpu as pltpu

_NCHUNKS = 8


def _copy_body(in_ref, out_ref, sems):
    rows = in_ref.shape[1]
    chunk = rows // _NCHUNKS
    copies = []
    for i in range(_NCHUNKS):
        sl = pl.ds(i * chunk, chunk)
        copies.append(
            pltpu.make_async_copy(
                in_ref.at[:, sl, :], out_ref.at[:, sl, :], sems.at[i]
            )
        )
    for c in copies:
        c.start()
    for c in copies:
        c.wait()


def kernel(inputs):
    return pl.pallas_call(
        _copy_body,
        out_shape=jax.ShapeDtypeStruct(inputs.shape, inputs.dtype),
        in_specs=[pl.BlockSpec(memory_space=pl.ANY)],
        out_specs=pl.BlockSpec(memory_space=pl.ANY),
        scratch_shapes=[pltpu.SemaphoreType.DMA((_NCHUNKS,))],
    )(inputs)


# speedup vs baseline: 49.0728x; 49.0728x over previous
import jax
from jax.experimental import pallas as pl
from jax.experimental.pallas import tpu as pltpu

_ROWS = 512


def _copy_body(in_ref, out_ref):
    out_ref[...] = in_ref[...]


def kernel(inputs):
    shape = inputs.shape
    flat = inputs.reshape(-1, shape[-1])
    n_rows, n_cols = flat.shape
    grid = (n_rows // _ROWS,)
    out = pl.pallas_call(
        _copy_body,
        out_shape=jax.ShapeDtypeStruct(flat.shape, flat.dtype),
        grid=grid,
        in_specs=[pl.BlockSpec((_ROWS, n_cols), lambda i: (i, 0))],
        out_specs=pl.BlockSpec((_ROWS, n_cols), lambda i: (i, 0)),
    )(flat)
    return out.reshape(shape)
